# baseline (speedup 1.0000x reference)
import jax
import jax.numpy as jnp
import numpy as np
from jax.experimental import pallas as pl
from jax.experimental.pallas import tpu as pltpu

_G = 64
_KV = 27
_NP = 50176


def _kernel_maps(coords, n):
    M = _G + 2
    c = coords.astype(jnp.int32) + 1
    keys = c[:, 0] * (M * M) + c[:, 1] * M + c[:, 2]
    order = jnp.argsort(keys)
    skeys = keys[order]
    offs = []
    for dx in (-1, 0, 1):
        for dy in (-1, 0, 1):
            for dz in (-1, 0, 1):
                offs.append(dx * M * M + dy * M + dz)
    offs = jnp.asarray(offs, jnp.int32)
    q = keys[None, :] + offs[:, None]
    pos = jnp.searchsorted(skeys, q)
    posc = jnp.clip(pos, 0, n - 1)
    found = skeys[posc] == q
    nbr = jnp.where(found, order[posc], n)
    return jnp.concatenate(
        [nbr, jnp.full((_KV, _NP - n), n, jnp.int32)], axis=1)


def _mm_acc_body(xg_ref, w_ref, o_ref):
    k = pl.program_id(1)

    @pl.when(k == 0)
    def _():
        o_ref[...] = jnp.zeros_like(o_ref)

    o_ref[...] += jnp.dot(xg_ref[0], w_ref[0],
                          preferred_element_type=jnp.float32)


def _conv_mm(xg, W, tr):
    K, n, cin = xg.shape
    cout = W.shape[2]
    return pl.pallas_call(
        _mm_acc_body,
        grid=(n // tr, K),
        in_specs=[
            pl.BlockSpec((1, tr, cin), lambda i, k: (k, i, 0)),
            pl.BlockSpec((1, cin, cout), lambda i, k: (k, 0, 0)),
        ],
        out_specs=pl.BlockSpec((tr, cout), lambda i, k: (i, 0)),
        out_shape=jax.ShapeDtypeStruct((n, cout), jnp.float32),
        compiler_params=pltpu.CompilerParams(
            dimension_semantics=("parallel", "arbitrary")),
    )(xg, W)


def _fan_mm_body(h_ref, u_ref, o_ref):
    o_ref[0] = jnp.dot(h_ref[...], u_ref[0],
                       preferred_element_type=jnp.float32)


def _fan_mm(h2, U, tr):
    n = h2.shape[0]
    K, cin, cout = U.shape
    return pl.pallas_call(
        _fan_mm_body,
        grid=(n // tr, K),
        in_specs=[
            pl.BlockSpec((tr, cin), lambda i, m: (i, 0)),
            pl.BlockSpec((1, cin, cout), lambda i, m: (m, 0, 0)),
        ],
        out_specs=pl.BlockSpec((1, tr, cout), lambda i, m: (m, i, 0)),
        out_shape=jax.ShapeDtypeStruct((K, n, cout), jnp.float32),
        compiler_params=pltpu.CompilerParams(
            dimension_semantics=("parallel", "arbitrary")),
    )(h2, U)


def kernel(x, coords, W1, W2, W3, W4):
    n = x.shape[0]
    nbr = _kernel_maps(coords, n)

    x_p = jnp.zeros((_NP, 2), x.dtype).at[:n].set(x)
    xg = jnp.stack([x_p[nbr[k]] for k in range(_KV)])
    Xg = jnp.zeros((_NP, 64), x.dtype).at[:, :2 * _KV].set(
        xg.transpose(1, 0, 2).reshape(_NP, 2 * _KV))

    Xgg = jnp.stack([Xg[nbr[k]] for k in range(_KV)])

    W1cat = jnp.zeros((64, 256), x.dtype).at[:2 * _KV].set(
        W1.reshape(2 * _KV, 256))
    V2 = jnp.einsum('ac,kcd->kad', W1cat, W2,
                    precision=jax.lax.Precision.HIGHEST)
    h2 = _conv_mm(Xgg, V2, tr=6272)

    W4cat = W4.transpose(1, 0, 2).reshape(256, 2 * _KV)
    U = jnp.zeros((_KV, 256, 64), x.dtype).at[:, :, :2 * _KV].set(
        jnp.einsum('kab,bc->kac', W3, W4cat,
                   precision=jax.lax.Precision.HIGHEST))
    T = _fan_mm(h2, U, tr=6272)

    Tf = T.reshape(_KV * _NP, 64)
    y = sum(Tf[nbr[m] + m * _NP] for m in range(_KV))

    Y2 = y[:, :2 * _KV].reshape(_NP * _KV, 2)
    out = sum(Y2[nbr[k] * _KV + k] for k in range(_KV))
    return out[:n]


# speedup vs baseline: 1.4093x; 1.4093x over previous
import functools

import jax
import jax.numpy as jnp
from jax import lax
from jax.experimental import pallas as pl
from jax.experimental.pallas import tpu as pltpu
from jax.experimental.pallas import tpu_sc as plsc

_G = 64
_KV = 27
_NW = 32
_CHUNK = 1568
_NP = _NW * _CHUNK
_BW = 112


def _kernel_maps(coords, n):
    M = _G + 2
    c = coords.astype(jnp.int32) + 1
    keys = c[:, 0] * (M * M) + c[:, 1] * M + c[:, 2]
    order = jnp.argsort(keys)
    skeys = keys[order]
    offs = []
    for dx in (-1, 0, 1):
        for dy in (-1, 0, 1):
            for dz in (-1, 0, 1):
                offs.append(dx * M * M + dy * M + dz)
    offs = jnp.asarray(offs, jnp.int32)
    q = keys[None, :] + offs[:, None]
    pos = jnp.searchsorted(skeys, q)
    posc = jnp.clip(pos, 0, n - 1)
    found = skeys[posc] == q
    nbr = jnp.where(found, order[posc], n)
    return jnp.concatenate(
        [nbr, jnp.full((_KV, _NP - n), n, jnp.int32)], axis=1)


def _sc_gather(table, idx, D, q):
    sub = 14 // q
    per_w = _KV * q
    nj = _NW * per_w
    idx3 = idx.reshape(_KV, _NW, _CHUNK).transpose(1, 0, 2).reshape(
        nj, sub, _BW)
    mesh = plsc.VectorSubcoreMesh(core_axis_name="c", subcore_axis_name="s")

    @functools.partial(
        pl.kernel, mesh=mesh,
        compiler_params=pltpu.CompilerParams(use_tc_tiling_on_sc=False),
        out_type=jax.ShapeDtypeStruct((nj, sub, _BW, D), jnp.float32),
        scratch_types=[
            pltpu.VMEM((2, sub, _BW), jnp.int32),
            pltpu.VMEM((2, sub, _BW, D), jnp.float32),
            pltpu.SemaphoreType.DMA,
            pltpu.SemaphoreType.DMA,
            pltpu.SemaphoreType.DMA,
            pltpu.SemaphoreType.DMA,
        ],
    )
    def body(table_hbm, idx_hbm, out_hbm, idx_v, rows_v, g0, g1, o0, o1):
        gsem = (g0, g1)
        osem = (o0, o1)
        w = lax.axis_index("s") * 2 + lax.axis_index("c")
        j0 = w * per_w

        def idx_load(t):
            pltpu.sync_copy(idx_hbm.at[j0 + t], idx_v.at[t % 2])

        def g_desc(t, j):
            sl = t % 2
            return pltpu.make_async_copy(
                table_hbm.at[idx_v.at[sl, j]], rows_v.at[sl, j], gsem[sl])

        def o_desc(t):
            sl = t % 2
            return pltpu.make_async_copy(
                rows_v.at[sl], out_hbm.at[j0 + t], osem[sl])

        idx_load(0)
        for j in range(sub):
            g_desc(0, j).start()
        if per_w > 1:
            idx_load(1)
            for j in range(sub):
                g_desc(1, j).start()
        for t in range(per_w):
            for j in range(sub):
                g_desc(t, j).wait()
            o_desc(t).start()
            if t + 2 < per_w:
                o_desc(t).wait()
                idx_load(t + 2)
                for j in range(sub):
                    g_desc(t + 2, j).start()
        for t in (max(per_w - 2, 0), per_w - 1):
            o_desc(t).wait()

    out = body(table, idx3)
    return out.reshape(_NW, _KV, _CHUNK, D)


def _mm_acc_body(xg_ref, w_ref, o_ref):
    k = pl.program_id(1)

    @pl.when(k == 0)
    def _():
        o_ref[...] = jnp.zeros_like(o_ref)

    o_ref[...] += jnp.dot(xg_ref[0], w_ref[0],
                          preferred_element_type=jnp.float32)


def _conv_mm(xg, W):
    nb, tr, cin = xg.shape
    cout = W.shape[2]
    return pl.pallas_call(
        _mm_acc_body,
        grid=(_NW, _KV),
        in_specs=[
            pl.BlockSpec((1, tr, cin), lambda i, k: (i * _KV + k, 0, 0)),
            pl.BlockSpec((1, cin, cout), lambda i, k: (k, 0, 0)),
        ],
        out_specs=pl.BlockSpec((tr, cout), lambda i, k: (i, 0)),
        out_shape=jax.ShapeDtypeStruct((_NP, cout), jnp.float32),
        compiler_params=pltpu.CompilerParams(
            dimension_semantics=("parallel", "arbitrary")),
    )(xg, W)


def _fan_mm_body(h_ref, u_ref, o_ref):
    o_ref[0] = jnp.dot(h_ref[...], u_ref[0],
                       preferred_element_type=jnp.float32)


def _fan_mm(h2, U, tr):
    n = h2.shape[0]
    K, cin, cout = U.shape
    return pl.pallas_call(
        _fan_mm_body,
        grid=(n // tr, K),
        in_specs=[
            pl.BlockSpec((tr, cin), lambda i, m: (i, 0)),
            pl.BlockSpec((1, cin, cout), lambda i, m: (m, 0, 0)),
        ],
        out_specs=pl.BlockSpec((1, tr, cout), lambda i, m: (m, i, 0)),
        out_shape=jax.ShapeDtypeStruct((K, n, cout), jnp.float32),
        compiler_params=pltpu.CompilerParams(
            dimension_semantics=("parallel", "arbitrary")),
    )(h2, U)


def kernel(x, coords, W1, W2, W3, W4):
    n = x.shape[0]
    nbr = _kernel_maps(coords, n)

    x_p = jnp.zeros((_NP, 16), x.dtype).at[:n, :2].set(x)
    GA = _sc_gather(x_p, nbr, 16, 1)
    Xg = jnp.zeros((_NP, 64), x.dtype).at[:, :2 * _KV].set(
        GA[:, :, :, :2].transpose(0, 2, 1, 3).reshape(_NP, 2 * _KV))

    GB = _sc_gather(Xg, nbr, 64, 2)

    W1cat = jnp.zeros((64, 256), x.dtype).at[:2 * _KV].set(
        W1.reshape(2 * _KV, 256))
    V2 = jnp.einsum('ac,kcd->kad', W1cat, W2,
                    precision=lax.Precision.HIGHEST)
    h2 = _conv_mm(GB.reshape(_NW * _KV, _CHUNK, 64), V2)

    W4cat = W4.transpose(1, 0, 2).reshape(256, 2 * _KV)
    U = jnp.zeros((_KV, 256, 64), x.dtype).at[:, :, :2 * _KV].set(
        jnp.einsum('kab,bc->kac', W3, W4cat,
                   precision=lax.Precision.HIGHEST))
    T = _fan_mm(h2, U, tr=_CHUNK)

    idxC = nbr + (jnp.arange(_KV, dtype=jnp.int32) * _NP)[:, None]
    GC = _sc_gather(T.reshape(_KV * _NP, 64), idxC, 64, 2)
    y = GC.sum(axis=1).reshape(_NP, 64)

    Y16 = jnp.zeros((_NP, _KV, 16), y.dtype).at[:, :, :2].set(
        y[:, :2 * _KV].reshape(_NP, _KV, 2)).reshape(_NP * _KV, 16)
    idxD = nbr * _KV + jnp.arange(_KV, dtype=jnp.int32)[:, None]
    GD = _sc_gather(Y16, idxD, 16, 1)
    out = GD[:, :, :, :2].sum(axis=1).reshape(_NP, 2)
    return out[:n]


# speedup vs baseline: 2.8791x; 2.0429x over previous
import functools

import jax
import jax.numpy as jnp
from jax import lax
from jax.experimental import pallas as pl
from jax.experimental.pallas import tpu as pltpu
from jax.experimental.pallas import tpu_sc as plsc

_G = 64
_KV = 27
_NW = 32
_CHUNK = 1568
_NP = _NW * _CHUNK
_BW = 112


def _kernel_maps(coords, n):
    M = _G + 2
    c = coords.astype(jnp.int32) + 1
    keys = c[:, 0] * (M * M) + c[:, 1] * M + c[:, 2]
    lut = jnp.full((M * M * M,), n, jnp.int32).at[keys].set(
        jnp.arange(n, dtype=jnp.int32))
    nbrs = []
    for dx in (-1, 0, 1):
        for dy in (-1, 0, 1):
            for dz in (-1, 0, 1):
                nbrs.append(lut[keys + (dx * M * M + dy * M + dz)])
    nbr = jnp.stack(nbrs)
    return jnp.concatenate(
        [nbr, jnp.full((_KV, _NP - n), n, jnp.int32)], axis=1)


def _sc_gather(table, idx, q):
    D = 128
    sub = 14 // q
    per_w = _KV * q
    nj = _NW * per_w
    idx3 = idx.reshape(_KV, _NW, _CHUNK).transpose(1, 0, 2).reshape(
        nj, sub, _BW)
    mesh = plsc.VectorSubcoreMesh(core_axis_name="c", subcore_axis_name="s")

    @functools.partial(
        pl.kernel, mesh=mesh,
        out_type=jax.ShapeDtypeStruct((nj, sub, _BW, D), jnp.float32),
        scratch_types=[
            pltpu.VMEM((2, sub, _BW), jnp.int32),
            pltpu.VMEM((2, sub, _BW, D), jnp.float32),
            pltpu.SemaphoreType.DMA,
            pltpu.SemaphoreType.DMA,
            pltpu.SemaphoreType.DMA,
            pltpu.SemaphoreType.DMA,
        ],
    )
    def body(table_hbm, idx_hbm, out_hbm, idx_v, rows_v, g0, g1, o0, o1):
        gsem = (g0, g1)
        osem = (o0, o1)
        w = lax.axis_index("s") * 2 + lax.axis_index("c")
        j0 = w * per_w

        def idx_load(t):
            pltpu.sync_copy(idx_hbm.at[j0 + t], idx_v.at[t % 2])

        def g_desc(t, j):
            sl = t % 2
            return pltpu.make_async_copy(
                table_hbm.at[idx_v.at[sl, j]], rows_v.at[sl, j], gsem[sl])

        def o_desc(t):
            sl = t % 2
            return pltpu.make_async_copy(
                rows_v.at[sl], out_hbm.at[j0 + t], osem[sl])

        idx_load(0)
        for j in range(sub):
            g_desc(0, j).start()
        if per_w > 1:
            idx_load(1)
            for j in range(sub):
                g_desc(1, j).start()
        for t in range(per_w):
            for j in range(sub):
                g_desc(t, j).wait()
            o_desc(t).start()
            if t + 2 < per_w:
                o_desc(t).wait()
                idx_load(t + 2)
                for j in range(sub):
                    g_desc(t + 2, j).start()
        for t in (max(per_w - 2, 0), per_w - 1):
            o_desc(t).wait()

    out = body(table, idx3)
    return out.reshape(_NW, _KV, _CHUNK, D)


def _mm_acc_body(xg_ref, w_ref, o_ref):
    k = pl.program_id(1)

    @pl.when(k == 0)
    def _():
        o_ref[...] = jnp.zeros_like(o_ref)

    o_ref[...] += jnp.dot(xg_ref[0], w_ref[0],
                          preferred_element_type=jnp.float32)


def _conv_mm(xg, W):
    nb, tr, cin = xg.shape
    cout = W.shape[2]
    return pl.pallas_call(
        _mm_acc_body,
        grid=(_NW, _KV),
        in_specs=[
            pl.BlockSpec((1, tr, cin), lambda i, k: (i * _KV + k, 0, 0)),
            pl.BlockSpec((1, cin, cout), lambda i, k: (k, 0, 0)),
        ],
        out_specs=pl.BlockSpec((tr, cout), lambda i, k: (i, 0)),
        out_shape=jax.ShapeDtypeStruct((_NP, cout), jnp.float32),
        compiler_params=pltpu.CompilerParams(
            dimension_semantics=("parallel", "arbitrary")),
    )(xg, W)


def _fan_mm_body(h_ref, u_ref, o_ref):
    o_ref[0] = jnp.dot(h_ref[...], u_ref[0],
                       preferred_element_type=jnp.float32)


def _fan_mm(h2, U, tr):
    n = h2.shape[0]
    K, cin, cout = U.shape
    return pl.pallas_call(
        _fan_mm_body,
        grid=(n // tr, K),
        in_specs=[
            pl.BlockSpec((tr, cin), lambda i, m: (i, 0)),
            pl.BlockSpec((1, cin, cout), lambda i, m: (m, 0, 0)),
        ],
        out_specs=pl.BlockSpec((1, tr, cout), lambda i, m: (m, i, 0)),
        out_shape=jax.ShapeDtypeStruct((K, n, cout), jnp.float32),
        compiler_params=pltpu.CompilerParams(
            dimension_semantics=("parallel", "arbitrary")),
    )(h2, U)


def kernel(x, coords, W1, W2, W3, W4):
    n = x.shape[0]
    nbr = _kernel_maps(coords, n)

    x_p = jnp.zeros((_NP, 2), x.dtype).at[:n].set(x)
    xg = jnp.stack([x_p[nbr[k]] for k in range(_KV)])
    Xg = jnp.zeros((_NP, 128), x.dtype).at[:, :2 * _KV].set(
        xg.transpose(1, 0, 2).reshape(_NP, 2 * _KV))

    GB = _sc_gather(Xg, nbr, 7)

    W1cat = jnp.zeros((128, 256), x.dtype).at[:2 * _KV].set(
        W1.reshape(2 * _KV, 256))
    V2 = jnp.einsum('ac,kcd->kad', W1cat, W2,
                    precision=lax.Precision.HIGHEST)
    h2 = _conv_mm(GB.reshape(_NW * _KV, _CHUNK, 128), V2)

    W4cat = W4.transpose(1, 0, 2).reshape(256, 2 * _KV)
    U = jnp.zeros((_KV, 256, 128), x.dtype).at[:, :, :2 * _KV].set(
        jnp.einsum('kab,bc->kac', W3, W4cat,
                   precision=lax.Precision.HIGHEST))
    T = _fan_mm(h2, U, tr=_CHUNK)

    idxC = nbr + (jnp.arange(_KV, dtype=jnp.int32) * _NP)[:, None]
    GC = _sc_gather(T.reshape(_KV * _NP, 128), idxC, 7)
    y = GC.sum(axis=1).reshape(_NP, 128)

    Y2 = y[:, :2 * _KV].reshape(_NP * _KV, 2)
    idxD = nbr * _KV + jnp.arange(_KV, dtype=jnp.int32)[:, None]
    out = sum(Y2[idxD[k]] for k in range(_KV))
    return out[:n]


# speedup vs baseline: 7.0474x; 2.4477x over previous
import functools

import jax
import jax.numpy as jnp
from jax import lax
from jax.experimental import pallas as pl
from jax.experimental.pallas import tpu as pltpu
from jax.experimental.pallas import tpu_sc as plsc

_G = 64
_KV = 27
_NW = 32
_CHUNK = 1568
_NP = _NW * _CHUNK
_BW = 112


def _kernel_maps(coords, n):
    M = _G + 2
    c = coords.astype(jnp.int32) + 1
    keys = c[:, 0] * (M * M) + c[:, 1] * M + c[:, 2]
    lut = jnp.full((M * M * M,), n, jnp.int32).at[keys].set(
        jnp.arange(n, dtype=jnp.int32))
    nbrs = []
    for dx in (-1, 0, 1):
        for dy in (-1, 0, 1):
            for dz in (-1, 0, 1):
                nbrs.append(lut[keys + (dx * M * M + dy * M + dz)])
    nbr = jnp.stack(nbrs)
    return jnp.concatenate(
        [nbr, jnp.full((_KV, _NP - n), n, jnp.int32)], axis=1)


def _sc_gather(table, idx, q):
    S, R, D = table.shape
    bb = _CHUNK // q
    per_w = _KV * q
    nj = _NW * per_w
    idx3 = idx.reshape(_KV, _NW, _CHUNK).transpose(1, 0, 2).reshape(
        nj, bb)
    mesh = plsc.VectorSubcoreMesh(core_axis_name="c", subcore_axis_name="s")

    @functools.partial(
        pl.kernel, mesh=mesh,
        compiler_params=pltpu.CompilerParams(use_tc_tiling_on_sc=False),
        out_type=jax.ShapeDtypeStruct((nj, bb, D), jnp.float32),
        scratch_types=[
            pltpu.VMEM((3, bb), jnp.int32),
            pltpu.VMEM((3, bb, D), jnp.float32),
            pltpu.SemaphoreType.DMA,
            pltpu.SemaphoreType.DMA,
            pltpu.SemaphoreType.DMA,
            pltpu.SemaphoreType.DMA,
            pltpu.SemaphoreType.DMA,
            pltpu.SemaphoreType.DMA,
        ],
    )
    def body(table_hbm, idx_hbm, out_hbm, idx_v, rows_v,
             g0, g1, g2, o0, o1, o2):
        gsem = (g0, g1, g2)
        osem = (o0, o1, o2)
        w = lax.axis_index("s") * 2 + lax.axis_index("c")
        j0 = w * per_w

        def idx_load(t):
            pltpu.sync_copy(idx_hbm.at[j0 + t], idx_v.at[t % 3])

        def g_desc(t):
            sl = t % 3
            slab = min(t // q, S - 1)
            return pltpu.make_async_copy(
                table_hbm.at[slab].at[idx_v.at[sl]], rows_v.at[sl], gsem[sl])

        def o_desc(t):
            sl = t % 3
            return pltpu.make_async_copy(
                rows_v.at[sl], out_hbm.at[j0 + t], osem[sl])

        idx_load(0)
        g_desc(0).start()
        idx_load(1)
        g_desc(1).start()
        for t in range(per_w):
            g_desc(t).wait()
            o_desc(t).start()
            if t + 2 < per_w:
                if t >= 1:
                    o_desc(t - 1).wait()
                idx_load(t + 2)
                g_desc(t + 2).start()
        for t in range(max(per_w - 3, 0), per_w):
            o_desc(t).wait()

    out = body(table, idx3)
    return out.reshape(_NW, _KV, _CHUNK, D)


def _mm_acc_body(xg_ref, w_ref, o_ref):
    k = pl.program_id(1)

    @pl.when(k == 0)
    def _():
        o_ref[...] = jnp.zeros_like(o_ref)

    o_ref[...] += jnp.dot(xg_ref[0], w_ref[0],
                          preferred_element_type=jnp.float32)


def _conv_mm(xg, W):
    nb, tr, cin = xg.shape
    cout = W.shape[2]
    return pl.pallas_call(
        _mm_acc_body,
        grid=(_NW, _KV),
        in_specs=[
            pl.BlockSpec((1, tr, cin), lambda i, k: (i * _KV + k, 0, 0)),
            pl.BlockSpec((1, cin, cout), lambda i, k: (k, 0, 0)),
        ],
        out_specs=pl.BlockSpec((tr, cout), lambda i, k: (i, 0)),
        out_shape=jax.ShapeDtypeStruct((_NP, cout), jnp.float32),
        compiler_params=pltpu.CompilerParams(
            dimension_semantics=("parallel", "arbitrary")),
    )(xg, W)


def _fan_mm_body(h_ref, u_ref, o_ref):
    o_ref[0] = jnp.dot(h_ref[...], u_ref[0],
                       preferred_element_type=jnp.float32)


def _fan_mm(h2, U, tr):
    n = h2.shape[0]
    K, cin, cout = U.shape
    return pl.pallas_call(
        _fan_mm_body,
        grid=(n // tr, K),
        in_specs=[
            pl.BlockSpec((tr, cin), lambda i, m: (i, 0)),
            pl.BlockSpec((1, cin, cout), lambda i, m: (m, 0, 0)),
        ],
        out_specs=pl.BlockSpec((1, tr, cout), lambda i, m: (m, i, 0)),
        out_shape=jax.ShapeDtypeStruct((K, n, cout), jnp.float32),
        compiler_params=pltpu.CompilerParams(
            dimension_semantics=("parallel", "arbitrary")),
    )(h2, U)


def kernel(x, coords, W1, W2, W3, W4):
    n = x.shape[0]
    nbr = _kernel_maps(coords, n)

    x_p = jnp.zeros((1, _NP, 16), x.dtype).at[0, :n, :2].set(x)
    GA = _sc_gather(x_p, nbr, 1)
    Xg = jnp.zeros((1, _NP, 64), x.dtype).at[0, :, :2 * _KV].set(
        GA[:, :, :, :2].transpose(0, 2, 1, 3).reshape(_NP, 2 * _KV))

    GB = _sc_gather(Xg, nbr, 4)

    W1cat = jnp.zeros((64, 256), x.dtype).at[:2 * _KV].set(
        W1.reshape(2 * _KV, 256))
    V2 = jnp.einsum('ac,kcd->kad', W1cat, W2,
                    precision=lax.Precision.HIGHEST)
    h2 = _conv_mm(GB.reshape(_NW * _KV, _CHUNK, 64), V2)

    W4cat = W4.transpose(1, 0, 2).reshape(256, 2 * _KV)
    U = jnp.zeros((_KV, 256, 64), x.dtype).at[:, :, :2 * _KV].set(
        jnp.einsum('kab,bc->kac', W3, W4cat,
                   precision=lax.Precision.HIGHEST))
    T = _fan_mm(h2, U, tr=_CHUNK)

    GC = _sc_gather(T, nbr, 4)
    y = GC.sum(axis=1).reshape(_NP, 64)

    Y16 = jnp.zeros((1, _NP * _KV, 16), y.dtype).at[0, :, :2].set(
        y[:, :2 * _KV].reshape(_NP * _KV, 2))
    idxD = nbr * _KV + jnp.arange(_KV, dtype=jnp.int32)[:, None]
    GD = _sc_gather(Y16, idxD, 1)
    out = GD[:, :, :, :2].sum(axis=1).reshape(_NP, 2)
    return out[:n]
